# baseline (speedup 1.0000x reference)
import functools

import jax
import jax.numpy as jnp
from jax.experimental import pallas as pl
from jax.experimental.pallas import tpu as pltpu


def _fused_meanpool_linear_kernel(x_ref, w_ref, b_ref, o_ref, *, inv_s):
    s = jnp.sum(x_ref[...], axis=1, dtype=jnp.float32)
    mean = s * inv_s
    y = jnp.dot(mean, w_ref[...], preferred_element_type=jnp.float32)
    o_ref[...] = (y + b_ref[...]).astype(o_ref.dtype)


def _pick_batch_tile(B, row_bytes, max_block_bytes):
    best_any = 1
    best_even = None
    for tb in range(1, B + 1):
        if B % tb != 0 or tb * row_bytes > max_block_bytes:
            continue
        best_any = tb
        if (B // tb) % 2 == 0:
            best_even = tb
    return best_even if best_even is not None else best_any


def kernel(x, weight, bias):
    B, S, C_in = x.shape
    C_out = weight.shape[-1]
    out_dtype = x.dtype
    inv_s = 1.0 / float(S)
    itemsize = x.dtype.itemsize

    row_bytes = S * C_in * itemsize
    tb = _pick_batch_tile(B, row_bytes, 8 * 1024 * 1024)
    nb = B // tb

    x_block_bytes = tb * row_bytes
    vmem_limit = int(min(2 * x_block_bytes + (8 << 20), 100 << 20))

    cost = pl.CostEstimate(
        flops=B * S * C_in + 2 * B * C_in * C_out,
        transcendentals=0,
        bytes_accessed=x.size * itemsize + weight.size * 4 + B * C_out * 4,
    )

    w = weight.astype(jnp.float32)
    b2d = bias.astype(jnp.float32).reshape(1, C_out)

    return pl.pallas_call(
        functools.partial(_fused_meanpool_linear_kernel, inv_s=inv_s),
        out_shape=jax.ShapeDtypeStruct((B, C_out), out_dtype),
        grid=(nb,),
        in_specs=[
            pl.BlockSpec((tb, S, C_in), lambda i: (i, 0, 0)),
            pl.BlockSpec((C_in, C_out), lambda i: (0, 0)),
            pl.BlockSpec((1, C_out), lambda i: (0, 0)),
        ],
        out_specs=pl.BlockSpec((tb, C_out), lambda i: (i, 0)),
        compiler_params=pltpu.CompilerParams(
            dimension_semantics=("parallel",),
            vmem_limit_bytes=vmem_limit,
        ),
        cost_estimate=cost,
    )(x, w, b2d)
